# Initial kernel scaffold; baseline (speedup 1.0000x reference)
import jax
import jax.numpy as jnp
from jax.experimental import pallas as pl


def kernel(x):
    raise NotImplementedError("write your pallas kernel here")


# speedup vs baseline: 4.6960x; 4.6960x over previous
import functools

import jax
import jax.numpy as jnp
from jax import lax
from jax.experimental import pallas as pl
from jax.experimental.pallas import tpu as pltpu
from jax.experimental.pallas import tpu_sc as plsc

_P = 0.12
_NC = 2
_NS = 16
_R = 4


def _jitter_indices(B, T):
    logits = jnp.log(jnp.array([_P / 2, 1 - _P, _P / 2], dtype=jnp.float32))
    key = jax.random.fold_in(jax.random.key(0), 1)
    idx = jax.random.categorical(key, logits, shape=(B, T)) - 1
    idx = idx.at[:, 0].set(jnp.clip(idx[:, 0], 0, 1))
    idx = idx.at[:, -1].set(jnp.clip(idx[:, -1], -1, 0))
    idx = idx + jnp.arange(T, dtype=idx.dtype)
    return idx.astype(jnp.int32)


def _sc_gather(B, C, T):
    L = 16
    G = C // _R
    mesh = plsc.VectorSubcoreMesh(core_axis_name="c", subcore_axis_name="s")

    def body(x_hbm, idx_hbm, out_hbm, idx_v, in_buf, out_buf,
             sin0, sin1, sout0, sout1):
        b = lax.axis_index("s") * _NC + lax.axis_index("c")
        pltpu.sync_copy(idx_hbm.at[b], idx_v)
        sin = (sin0, sin1)
        sout = (sout0, sout1)

        def in_copy(g, s):
            return pltpu.make_async_copy(
                x_hbm.at[b, pl.ds(g * _R, _R), :], in_buf.at[s], sin[s])

        def out_copy(g, s):
            return pltpu.make_async_copy(
                out_buf.at[s], out_hbm.at[b, pl.ds(g * _R, _R), :], sout[s])

        in_copy(0, 0).start()
        in_copy(1, 1).start()

        def group_pair(i, carry):
            g0 = i * 2
            for s in (0, 1):
                g = g0 + s
                in_copy(g, s).wait()

                @pl.when(g >= 2)
                def _wait_out():
                    out_copy(g - 2, s).wait()

                def chunk(ci, c2):
                    src = idx_v[pl.ds(ci * L, L)]
                    i_s = jnp.full((L,), s, jnp.int32)
                    for r in range(_R):
                        i_r = jnp.full((L,), r, jnp.int32)
                        v = plsc.load_gather(in_buf, [i_s, i_r, src])
                        out_buf[s, r, pl.ds(ci * L, L)] = v
                    return c2

                lax.fori_loop(0, T // L, chunk, 0, unroll=2)
                out_copy(g, s).start()

                @pl.when(g + 2 < G)
                def _next_in():
                    in_copy(g + 2, s).start()
            return carry

        lax.fori_loop(0, G // 2, group_pair, 0)
        out_copy(G - 2, 0).wait()
        out_copy(G - 1, 1).wait()

    return pl.kernel(
        body,
        out_type=jax.ShapeDtypeStruct((B, C, T), jnp.float32),
        mesh=mesh,
        compiler_params=pltpu.CompilerParams(needs_layout_passes=False),
        scratch_types=[
            pltpu.VMEM((T,), jnp.int32),
            pltpu.VMEM((2, _R, T), jnp.float32),
            pltpu.VMEM((2, _R, T), jnp.float32),
            pltpu.SemaphoreType.DMA,
            pltpu.SemaphoreType.DMA,
            pltpu.SemaphoreType.DMA,
            pltpu.SemaphoreType.DMA,
        ],
    )


@functools.lru_cache(maxsize=None)
def _build(B, C, T):
    return _sc_gather(B, C, T)


def kernel(x):
    B, C, T = x.shape
    idx = _jitter_indices(B, T)
    return _build(B, C, T)(x, idx)


# speedup vs baseline: 14.2889x; 3.0428x over previous
import functools

import jax
import jax.numpy as jnp
from jax import lax
from jax.experimental import pallas as pl
from jax.experimental.pallas import tpu as pltpu
from jax.experimental.pallas import tpu_sc as plsc

_P = 0.12
_NC = 2
_NS = 16
_R = 4


def _jitter_indices(B, T):
    logits = jnp.log(jnp.array([_P / 2, 1 - _P, _P / 2], dtype=jnp.float32))
    key = jax.random.fold_in(jax.random.key(0), 1)
    idx = jax.random.categorical(key, logits, shape=(B, T)) - 1
    idx = idx.at[:, 0].set(jnp.clip(idx[:, 0], 0, 1))
    idx = idx.at[:, -1].set(jnp.clip(idx[:, -1], -1, 0))
    idx = idx + jnp.arange(T, dtype=idx.dtype)
    return idx.astype(jnp.int32)


def _sc_gather(B, C, T):
    L = 16
    G = C // _R
    mesh = plsc.VectorSubcoreMesh(core_axis_name="c", subcore_axis_name="s")

    def body(x_hbm, idx_hbm, out_hbm, idx_v, *rest):
        bufs, sems = rest[:4 * _R], rest[4 * _R:]
        ins = (bufs[0:_R], bufs[_R:2 * _R])
        outs = (bufs[2 * _R:3 * _R], bufs[3 * _R:4 * _R])
        sin = sems[0:2]
        sout = sems[2:4]

        b = lax.axis_index("s") * _NC + lax.axis_index("c")
        pltpu.sync_copy(idx_hbm.at[b], idx_v)

        def in_copy(g, s, r):
            return pltpu.make_async_copy(
                x_hbm.at[b, g * _R + r], ins[s][r], sin[s])

        def out_copy(g, s, r):
            return pltpu.make_async_copy(
                outs[s][r], out_hbm.at[b, g * _R + r], sout[s])

        for r in range(_R):
            in_copy(0, 0, r).start()
        for r in range(_R):
            in_copy(1, 1, r).start()

        def group_pair(i, carry):
            g0 = i * 2
            for s in (0, 1):
                g = g0 + s
                for r in range(_R):
                    in_copy(g, s, r).wait()

                @pl.when(g >= 2)
                def _wait_out():
                    for r in range(_R):
                        out_copy(g - 2, s, r).wait()

                @plsc.parallel_loop(0, T, step=L, unroll=4)
                def _chunk(t0):
                    src = idx_v[pl.ds(t0, L)]
                    for r in range(_R):
                        v = plsc.load_gather(ins[s][r], [src])
                        outs[s][r][pl.ds(t0, L)] = v
                for r in range(_R):
                    out_copy(g, s, r).start()

                @pl.when(g + 2 < G)
                def _next_in():
                    for r in range(_R):
                        in_copy(g + 2, s, r).start()
            return carry

        lax.fori_loop(0, G // 2, group_pair, 0)
        for r in range(_R):
            out_copy(G - 2, 0, r).wait()
        for r in range(_R):
            out_copy(G - 1, 1, r).wait()

    return pl.kernel(
        body,
        out_type=jax.ShapeDtypeStruct((B, C, T), jnp.float32),
        mesh=mesh,
        compiler_params=pltpu.CompilerParams(needs_layout_passes=False),
        scratch_types=(
            [pltpu.VMEM((T,), jnp.int32)]
            + [pltpu.VMEM((T,), jnp.float32) for _ in range(4 * _R)]
            + [pltpu.SemaphoreType.DMA for _ in range(4)]
        ),
    )


@functools.lru_cache(maxsize=None)
def _build(B, C, T):
    return _sc_gather(B, C, T)


def kernel(x):
    B, C, T = x.shape
    idx = _jitter_indices(B, T)
    return _build(B, C, T)(x, idx)


# speedup vs baseline: 14.6180x; 1.0230x over previous
import functools

import jax
import jax.numpy as jnp
from jax import lax
from jax.experimental import pallas as pl
from jax.experimental.pallas import tpu as pltpu
from jax.experimental.pallas import tpu_sc as plsc

_P = 0.12
_NC = 2
_NS = 16
_R = 4


def _jitter_indices(B, T):
    logits = jnp.log(jnp.array([_P / 2, 1 - _P, _P / 2], dtype=jnp.float32))
    key = jax.random.fold_in(jax.random.key(0), 1)
    idx = jax.random.categorical(key, logits, shape=(B, T)) - 1
    idx = idx.at[:, 0].set(jnp.clip(idx[:, 0], 0, 1))
    idx = idx.at[:, -1].set(jnp.clip(idx[:, -1], -1, 0))
    idx = idx + jnp.arange(T, dtype=idx.dtype)
    return idx.astype(jnp.int32)


def _sc_gather(B, C, T):
    L = 16
    G = C // _R
    mesh = plsc.VectorSubcoreMesh(core_axis_name="c", subcore_axis_name="s")

    S = 3

    def body(x_hbm, idx_hbm, out_hbm, idx_v, *rest):
        bufs, sems = rest[:2 * S * _R], rest[2 * S * _R:]
        ins = tuple(bufs[i * _R:(i + 1) * _R] for i in range(S))
        outs = tuple(bufs[(S + i) * _R:(S + i + 1) * _R] for i in range(S))
        sin = sems[0:S]
        sout = sems[S:2 * S]

        b = lax.axis_index("s") * _NC + lax.axis_index("c")
        pltpu.sync_copy(idx_hbm.at[b], idx_v)

        def in_copy(g, s, r):
            return pltpu.make_async_copy(
                x_hbm.at[b, g * _R + r], ins[s][r], sin[s])

        def out_copy(g, s, r):
            return pltpu.make_async_copy(
                outs[s][r], out_hbm.at[b, g * _R + r], sout[s])

        for s in range(S):
            for r in range(_R):
                in_copy(s, s, r).start()

        def step(g, s):
            g = jnp.int32(g)
            for r in range(_R):
                in_copy(g, s, r).wait()

            @pl.when(g >= S)
            def _wait_out():
                for r in range(_R):
                    out_copy(g - S, s, r).wait()

            @plsc.parallel_loop(0, T, step=L, unroll=4)
            def _chunk(t0):
                src = idx_v[pl.ds(t0, L)]
                for r in range(_R):
                    v = plsc.load_gather(ins[s][r], [src])
                    outs[s][r][pl.ds(t0, L)] = v
            for r in range(_R):
                out_copy(g, s, r).start()

            @pl.when(g + S < G)
            def _next_in():
                for r in range(_R):
                    in_copy(g + S, s, r).start()

        def group_block(i, carry):
            for s in range(S):
                step(i * S + s, s)
            return carry

        lax.fori_loop(0, G // S, group_block, 0)
        for g in range((G // S) * S, G):
            step(g, g % S)
        for g in range(G - S, G):
            for r in range(_R):
                out_copy(g, g % S, r).wait()

    return pl.kernel(
        body,
        out_type=jax.ShapeDtypeStruct((B, C, T), jnp.float32),
        mesh=mesh,
        compiler_params=pltpu.CompilerParams(needs_layout_passes=False),
        scratch_types=(
            [pltpu.VMEM((T,), jnp.int32)]
            + [pltpu.VMEM((T,), jnp.float32) for _ in range(2 * 3 * _R)]
            + [pltpu.SemaphoreType.DMA for _ in range(2 * 3)]
        ),
    )


@functools.lru_cache(maxsize=None)
def _build(B, C, T):
    return _sc_gather(B, C, T)


def kernel(x):
    B, C, T = x.shape
    idx = _jitter_indices(B, T)
    return _build(B, C, T)(x, idx)


# speedup vs baseline: 14.6195x; 1.0001x over previous
import functools

import jax
import jax.numpy as jnp
from jax import lax
from jax.experimental import pallas as pl
from jax.experimental.pallas import tpu as pltpu
from jax.experimental.pallas import tpu_sc as plsc

_P = 0.12
_NC = 2
_NS = 16
_R = 4


def _jitter_indices(B, T):
    logits = jnp.log(jnp.array([_P / 2, 1 - _P, _P / 2], dtype=jnp.float32))
    key = jax.random.fold_in(jax.random.key(0), 1)
    idx = jax.random.categorical(key, logits, shape=(B, T)) - 1
    idx = idx.at[:, 0].set(jnp.clip(idx[:, 0], 0, 1))
    idx = idx.at[:, -1].set(jnp.clip(idx[:, -1], -1, 0))
    idx = idx + jnp.arange(T, dtype=idx.dtype)
    return idx.astype(jnp.int32)


def _sc_gather(B, C, T):
    L = 16
    G = C // _R
    mesh = plsc.VectorSubcoreMesh(core_axis_name="c", subcore_axis_name="s")

    S = 3

    def body(x_hbm, idx_hbm, out_hbm, idx_v, *rest):
        bufs, sems = rest[:2 * S * _R], rest[2 * S * _R:]
        ins = tuple(bufs[i * _R:(i + 1) * _R] for i in range(S))
        outs = tuple(bufs[(S + i) * _R:(S + i + 1) * _R] for i in range(S))
        sin = sems[0:S]
        sout = sems[S:2 * S]

        b = lax.axis_index("s") * _NC + lax.axis_index("c")
        pltpu.sync_copy(idx_hbm.at[b], idx_v)

        def in_copy(g, s, r):
            return pltpu.make_async_copy(
                x_hbm.at[b, g * _R + r], ins[s][r], sin[s])

        def out_copy(g, s, r):
            return pltpu.make_async_copy(
                outs[s][r], out_hbm.at[b, g * _R + r], sout[s])

        for s in range(S):
            for r in range(_R):
                in_copy(s, s, r).start()

        def step(g, s):
            g = jnp.int32(g)
            for r in range(_R):
                in_copy(g, s, r).wait()

            @pl.when(g >= S)
            def _wait_out():
                for r in range(_R):
                    out_copy(g - S, s, r).wait()

            @plsc.parallel_loop(0, T, step=L, unroll=4)
            def _chunk(t0):
                src = idx_v[pl.ds(t0, L)]
                for r in range(_R):
                    v = plsc.load_gather(ins[s][r], [src])
                    outs[s][r][pl.ds(t0, L)] = v
            for r in range(_R):
                out_copy(g, s, r).start()

            @pl.when(g + S < G)
            def _next_in():
                for r in range(_R):
                    in_copy(g + S, s, r).start()

        def group_block(i, carry):
            for s in range(S):
                step(i * S + s, s)
            return carry

        lax.fori_loop(0, G // S, group_block, 0)
        for g in range((G // S) * S, G):
            step(g, g % S)
        for g in range(G - S, G):
            for r in range(_R):
                out_copy(g, g % S, r).wait()

    return pl.kernel(
        body,
        out_type=jax.ShapeDtypeStruct((B, C, T), jnp.float32),
        mesh=mesh,
        compiler_params=pltpu.CompilerParams(
            needs_layout_passes=False, skip_device_barrier=True),
        scratch_types=(
            [pltpu.VMEM((T,), jnp.int32)]
            + [pltpu.VMEM((T,), jnp.float32) for _ in range(2 * 3 * _R)]
            + [pltpu.SemaphoreType.DMA for _ in range(2 * 3)]
        ),
    )


@functools.lru_cache(maxsize=None)
def _build(B, C, T):
    return _sc_gather(B, C, T)


def kernel(x):
    B, C, T = x.shape
    idx = _jitter_indices(B, T)
    return _build(B, C, T)(x, idx)
